# baseline (speedup 1.0000x reference)
import functools

import jax
import jax.numpy as jnp
import numpy as np
from jax import lax
from jax.experimental import pallas as pl
from jax.experimental.pallas import tpu as pltpu
from jax.experimental.pallas import tpu_sc as plsc

B = 16384
F = 26
FIELD = 38462
NC = 2
NS = 16
NW = NC * NS
BPW = B // NW
IPW = BPW * F
GROW = 128
NROW = IPW // GROW
CHUNKS = BPW // 16


def _body(xf_hbm, wf_hbm, tidx_hbm, offs_hbm, bias_hbm, out_hbm,
          idxv, ids, vals, outv, bv, sem):
    wid = lax.axis_index("s") * NC + lax.axis_index("c")
    base = wid * IPW

    pltpu.sync_copy(tidx_hbm, idxv)
    pltpu.sync_copy(bias_hbm, bv)

    def shift(c, _):
        s = c * 16
        idxv[pl.ds(s, 16)] = idxv[pl.ds(s, 16)] + base
        return _
    lax.fori_loop(0, IPW // 16, shift, None)

    def fire_x(j, _):
        pltpu.async_copy(
            xf_hbm.at[idxv.at[pl.ds(j * GROW, GROW)]],
            ids.at[pl.ds(j * GROW, GROW)],
            sem)
        return _
    lax.fori_loop(0, NROW, fire_x, None)
    pltpu.make_async_copy(xf_hbm.at[pl.ds(0, IPW)], ids, sem).wait()

    pltpu.sync_copy(offs_hbm, idxv)

    def build(c, _):
        s = c * 16
        idxv[pl.ds(s, 16)] = idxv[pl.ds(s, 16)] + ids[pl.ds(s, 16)]
        return _
    lax.fori_loop(0, IPW // 16, build, None)

    def fire_w(j, _):
        pltpu.async_copy(
            wf_hbm.at[idxv.at[pl.ds(j * GROW, GROW)]],
            vals.at[pl.ds(j * GROW, GROW)],
            sem)
        return _
    lax.fori_loop(0, NROW, fire_w, None)
    pltpu.make_async_copy(wf_hbm.at[pl.ds(0, IPW)], vals, sem).wait()

    bias_v = bv[...]

    def reduce(c, _):
        s = c * 16
        acc = bias_v
        for f in range(F):
            acc = acc + vals[pl.ds(f * BPW + s, 16)]
        res = 1.0 / (1.0 + jnp.exp(-acc))
        outv[pl.ds(s, 16)] = res
        return _
    lax.fori_loop(0, CHUNKS, reduce, None)

    pltpu.sync_copy(outv, out_hbm.at[pl.ds(wid * BPW, BPW)])


def kernel(x, W, bias):
    xf = x.astype(jnp.int32).reshape(-1)
    wf = W.reshape(-1)
    tpat = (np.arange(BPW, dtype=np.int32)[None, :] * F
            + np.arange(F, dtype=np.int32)[:, None]).reshape(-1)
    tidx = jnp.asarray(tpat)
    offs = jnp.asarray(np.repeat(
        np.arange(F, dtype=np.int32) * FIELD, BPW))
    bias16 = jnp.broadcast_to(bias.astype(jnp.float32), (16,))

    mesh = plsc.VectorSubcoreMesh(core_axis_name="c", subcore_axis_name="s")
    run = functools.partial(
        pl.kernel,
        mesh=mesh,
        out_type=jax.ShapeDtypeStruct((B,), jnp.float32),
        scratch_types=[
            pltpu.VMEM((IPW,), jnp.int32),
            pltpu.VMEM((IPW,), jnp.int32),
            pltpu.VMEM((IPW,), jnp.float32),
            pltpu.VMEM((BPW,), jnp.float32),
            pltpu.VMEM((16,), jnp.float32),
            pltpu.SemaphoreType.DMA,
        ],
    )(_body)
    return run(xf, wf, tidx, offs, bias16)


# speedup vs baseline: 1.2102x; 1.2102x over previous
import functools

import jax
import jax.numpy as jnp
import numpy as np
from jax import lax
from jax.experimental import pallas as pl
from jax.experimental.pallas import tpu as pltpu
from jax.experimental.pallas import tpu_sc as plsc

B = 16384
F = 26
FIELD = 38462
NC = 2
NS = 16
NW = NC * NS
BPW = B // NW
IPW = BPW * F
GROW = 128
NROW = IPW // GROW
CHUNKS = BPW // 16


def _body(xf_hbm, wf_hbm, offs_hbm, bias_hbm, out_hbm,
          xv, idxv, vals, outv, bv, sem):
    wid = lax.axis_index("s") * NC + lax.axis_index("c")
    base = wid * IPW

    pltpu.sync_copy(xf_hbm.at[pl.ds(base, IPW)], xv)
    pltpu.sync_copy(offs_hbm, idxv)
    pltpu.sync_copy(bias_hbm, bv)

    def build(c, _):
        s = c * 16
        idxv[pl.ds(s, 16)] = idxv[pl.ds(s, 16)] + xv[pl.ds(s, 16)]
        return _
    lax.fori_loop(0, IPW // 16, build, None)

    def fire(j, _):
        pltpu.async_copy(
            wf_hbm.at[idxv.at[pl.ds(j * GROW, GROW)]],
            vals.at[pl.ds(j * GROW, GROW)],
            sem)
        return _
    lax.fori_loop(0, NROW, fire, None)
    pltpu.make_async_copy(wf_hbm.at[pl.ds(0, IPW)], vals, sem).wait()

    bias_v = bv[...]
    iota_f = lax.broadcasted_iota(jnp.int32, (16,), 0) * F

    def reduce(c, _):
        s = c * 16
        p0 = c * (16 * F) + iota_f
        acc = bias_v
        for f in range(F):
            acc = acc + plsc.load_gather(vals, [p0 + f])
        res = 1.0 / (1.0 + jnp.exp(-acc))
        outv[pl.ds(s, 16)] = res
        return _
    lax.fori_loop(0, CHUNKS, reduce, None)

    pltpu.sync_copy(outv, out_hbm.at[pl.ds(wid * BPW, BPW)])


def kernel(x, W, bias):
    xf = x.astype(jnp.int32).reshape(-1)
    wf = W.reshape(-1)
    offs = jnp.asarray(
        np.tile(np.arange(F, dtype=np.int32) * FIELD, BPW))
    bias16 = jnp.broadcast_to(bias.astype(jnp.float32), (16,))

    mesh = plsc.VectorSubcoreMesh(core_axis_name="c", subcore_axis_name="s")
    run = functools.partial(
        pl.kernel,
        mesh=mesh,
        out_type=jax.ShapeDtypeStruct((B,), jnp.float32),
        compiler_params=pltpu.CompilerParams(needs_layout_passes=False),
        scratch_types=[
            pltpu.VMEM((IPW,), jnp.int32),
            pltpu.VMEM((IPW,), jnp.int32),
            pltpu.VMEM((IPW,), jnp.float32),
            pltpu.VMEM((BPW,), jnp.float32),
            pltpu.VMEM((16,), jnp.float32),
            pltpu.SemaphoreType.DMA,
        ],
    )(_body)
    return run(xf, wf, offs, bias16)


# speedup vs baseline: 1.4686x; 1.2136x over previous
import functools

import jax
import jax.numpy as jnp
from jax import lax
from jax.experimental import pallas as pl
from jax.experimental.pallas import tpu as pltpu
from jax.experimental.pallas import tpu_sc as plsc

B = 16384
F = 26
FIELD = 38462
OFFS = [f * FIELD for f in range(F)]
NC = 2
NS = 16
NW = NC * NS
BPW = B // NW
IPW = BPW * F
GROW = 128
NROW = IPW // GROW
CHUNKS = BPW // 16


def _body(xt_hbm, wf_hbm, bias_hbm, out_hbm,
          xv, idxv, vals, outv, bv, sem):
    wid = lax.axis_index("s") * NC + lax.axis_index("c")
    base = wid * BPW

    pltpu.sync_copy(xt_hbm.at[:, pl.ds(base, BPW)], xv)
    pltpu.sync_copy(bias_hbm, bv)

    def build(c, _):
        s = c * 16
        for f in range(F):
            idxv[pl.ds(f * BPW + s, 16)] = xv[f, pl.ds(s, 16)] + OFFS[f]
        return _
    lax.fori_loop(0, CHUNKS, build, None)

    def fire(j, _):
        pltpu.async_copy(
            wf_hbm.at[idxv.at[pl.ds(j * GROW, GROW)]],
            vals.at[pl.ds(j * GROW, GROW)],
            sem)
        return _
    lax.fori_loop(0, NROW, fire, None)
    pltpu.make_async_copy(wf_hbm.at[pl.ds(0, IPW)], vals, sem).wait()

    bias_v = bv[...]

    def reduce(c, _):
        s = c * 16
        acc = bias_v
        for f in range(F):
            acc = acc + vals[pl.ds(f * BPW + s, 16)]
        res = 1.0 / (1.0 + jnp.exp(-acc))
        outv[pl.ds(s, 16)] = res
        return _
    lax.fori_loop(0, CHUNKS, reduce, None)

    pltpu.sync_copy(outv, out_hbm.at[pl.ds(base, BPW)])


def kernel(x, W, bias):
    xt = x.astype(jnp.int32).T
    wf = W.T.reshape(-1)
    bias16 = jnp.broadcast_to(bias.astype(jnp.float32), (16,))

    mesh = plsc.VectorSubcoreMesh(core_axis_name="c", subcore_axis_name="s")
    run = functools.partial(
        pl.kernel,
        mesh=mesh,
        out_type=jax.ShapeDtypeStruct((B,), jnp.float32),
        scratch_types=[
            pltpu.VMEM((F, BPW), jnp.int32),
            pltpu.VMEM((IPW,), jnp.int32),
            pltpu.VMEM((IPW,), jnp.float32),
            pltpu.VMEM((BPW,), jnp.float32),
            pltpu.VMEM((16,), jnp.float32),
            pltpu.SemaphoreType.DMA,
        ],
    )(_body)
    return run(xt, wf, bias16)
